# baseline (speedup 1.0000x reference)
import functools

import jax
import jax.numpy as jnp
from jax import lax
from jax.experimental import pallas as pl
from jax.experimental.pallas import tpu as pltpu
from jax.experimental.pallas import tpu_sc as plsc

N_NODES = 10000
N_EDGES = 320000
D = 128
L = 16
NC = 2
NS = 16
NW = NC * NS
E_PER_W = N_EDGES // NW
CHUNK = 80
NCHUNK = E_PER_W // CHUNK
GROUPS = CHUNK // L
NBUF = 8


def _lane_shuffle(v, idx):
    dnums = lax.GatherDimensionNumbers(
        offset_dims=(), collapsed_slice_dims=(0,), start_index_map=(0,))
    return lax.gather(v, idx[:, None], dnums, slice_sizes=(1,),
                      mode=lax.GatherScatterMode.PROMISE_IN_BOUNDS)


def _edge_dot(srows_v, drows_v, row):
    acc = None
    for j in range(D // 32):
        s2 = plsc.bitcast(srows_v[row, pl.ds(j * L, L)], jnp.bfloat16)
        d2 = plsc.bitcast(drows_v[row, pl.ds(j * L, L)], jnp.bfloat16)
        sa, sb = plsc.unpack(s2, format=plsc.PackFormat.INTERLEAVED)
        da, db = plsc.unpack(d2, format=plsc.PackFormat.INTERLEAVED)
        t = sa * da + sb * db
        acc = t if acc is None else acc + t
    return acc


def _sc_body(nf_hbm, src_hbm, dst_hbm, out_hbm, mins_hbm, maxs_hbm,
             sidx_v, didx_v, srows_v, drows_v, labels_v, mm_v,
             sem_s, sem_d):
    sid = lax.axis_index("s")
    wid = sid * NC + lax.axis_index("c")
    base = wid * E_PER_W

    pltpu.sync_copy(src_hbm.at[pl.ds(base, E_PER_W)], sidx_v)
    pltpu.sync_copy(dst_hbm.at[pl.ds(base, E_PER_W)], didx_v)

    def start_gather(c):
        b = c % NBUF
        pltpu.async_copy(
            nf_hbm.at[sidx_v.at[pl.ds(c * CHUNK, CHUNK)]],
            srows_v.at[b], sem_s)
        pltpu.async_copy(
            nf_hbm.at[didx_v.at[pl.ds(c * CHUNK, CHUNK)]],
            drows_v.at[b], sem_d)

    for cc in range(NBUF - 1):
        start_gather(cc)

    def chunk_body(c, carry):
        mn, mx = carry
        b = c % NBUF
        pltpu.make_async_copy(
            nf_hbm.at[sidx_v.at[pl.ds(c * CHUNK, CHUNK)]],
            srows_v.at[b], sem_s).wait()
        pltpu.make_async_copy(
            nf_hbm.at[didx_v.at[pl.ds(c * CHUNK, CHUNK)]],
            drows_v.at[b], sem_d).wait()

        @pl.when(c < NCHUNK - (NBUF - 1))
        def _():
            start_gather(c + NBUF - 1)

        lanes = lax.iota(jnp.int32, L)
        for g in range(GROUPS):
            lab = jnp.zeros((L,), jnp.float32)
            for e in range(L):
                acc = _edge_dot(srows_v.at[b], drows_v.at[b], g * L + e)
                for k in (8, 4, 2, 1):
                    acc = acc + _lane_shuffle(acc, lanes ^ k)
                lab = jnp.where(lanes == e, acc, lab)
            mn = jnp.minimum(mn, lab)
            mx = jnp.maximum(mx, lab)
            labels_v[pl.ds(c * CHUNK + g * L, L)] = lab
        return mn, mx

    init = (jnp.full((L,), jnp.inf, jnp.float32),
            jnp.full((L,), -jnp.inf, jnp.float32))
    mn, mx = lax.fori_loop(0, NCHUNK, chunk_body, init)

    mm_v[0, :] = mn
    mm_v[1, :] = mx
    pltpu.sync_copy(labels_v, out_hbm.at[pl.ds(base, E_PER_W)])
    pltpu.sync_copy(mm_v.at[0], mins_hbm.at[wid])
    pltpu.sync_copy(mm_v.at[1], maxs_hbm.at[wid])


def _pack_body(nf_ref, tbl_ref):
    u = lax.bitcast_convert_type(nf_ref[...], jnp.int32)
    r = u + 0x7FFF + (lax.shift_right_logical(u, 16) & 1)
    t = lax.shift_right_logical(r, 16)
    lo = t[:, : D // 2]
    hi = t[:, D // 2:]
    tbl_ref[...] = lax.shift_left(hi, 16) | lo


def _pack_table(nf):
    return pl.pallas_call(
        _pack_body,
        out_shape=jax.ShapeDtypeStruct((N_NODES, D // 2), jnp.int32),
    )(nf)


@jax.jit
def _sc_dot(nf_pack, src, dst):
    mesh = plsc.VectorSubcoreMesh(core_axis_name="c", subcore_axis_name="s")
    k = pl.kernel(
        _sc_body,
        out_type=(
            jax.ShapeDtypeStruct((N_EDGES,), jnp.float32),
            jax.ShapeDtypeStruct((NW, L), jnp.float32),
            jax.ShapeDtypeStruct((NW, L), jnp.float32),
        ),
        mesh=mesh,
        compiler_params=pltpu.CompilerParams(
            needs_layout_passes=False, use_tc_tiling_on_sc=False),
        scratch_types=[
            pltpu.VMEM((E_PER_W,), jnp.int32),
            pltpu.VMEM((E_PER_W,), jnp.int32),
            pltpu.VMEM((NBUF, CHUNK, D // 2), jnp.int32),
            pltpu.VMEM((NBUF, CHUNK, D // 2), jnp.int32),
            pltpu.VMEM((E_PER_W,), jnp.float32),
            pltpu.VMEM((2, L), jnp.float32),
            pltpu.SemaphoreType.DMA,
            pltpu.SemaphoreType.DMA,
        ],
    )
    return k(nf_pack, src, dst)


def _norm_body(lab_ref, mins_ref, maxs_ref, out_ref):
    mn = jnp.min(mins_ref[...])
    mx = jnp.max(maxs_ref[...])
    out_ref[...] = (lab_ref[...] - mn) / (mx - mn)


def _normalize(labels, mins, maxs):
    lab2 = labels.reshape(N_EDGES // D, D)
    return pl.pallas_call(
        _norm_body,
        out_shape=jax.ShapeDtypeStruct((N_EDGES // D, D), jnp.float32),
    )(lab2, mins.reshape(4, D), maxs.reshape(4, D))


def kernel(nf, edge_index):
    nf_pack = _pack_table(nf)
    labels, mins, maxs = _sc_dot(nf_pack, edge_index[0], edge_index[1])
    out = _normalize(labels, mins, maxs)
    return out.reshape(N_EDGES, 1)


# speedup vs baseline: 1.0540x; 1.0540x over previous
import functools

import jax
import jax.numpy as jnp
from jax import lax
from jax.experimental import pallas as pl
from jax.experimental.pallas import tpu as pltpu
from jax.experimental.pallas import tpu_sc as plsc

N_NODES = 10000
N_EDGES = 320000
D = 128
L = 16
NC = 2
NS = 16
NW = NC * NS
E_PER_W = N_EDGES // NW
CHUNK = 80
NCHUNK = E_PER_W // CHUNK
GROUPS = CHUNK // L
NBUF = 8


def _lane_shuffle(v, idx):
    dnums = lax.GatherDimensionNumbers(
        offset_dims=(), collapsed_slice_dims=(0,), start_index_map=(0,))
    return lax.gather(v, idx[:, None], dnums, slice_sizes=(1,),
                      mode=lax.GatherScatterMode.PROMISE_IN_BOUNDS)


def _edge_dot(srows_v, drows_v, row):
    acc = None
    for j in range(D // 32):
        s2 = plsc.bitcast(srows_v[row, pl.ds(j * L, L)], jnp.bfloat16)
        d2 = plsc.bitcast(drows_v[row, pl.ds(j * L, L)], jnp.bfloat16)
        sa, sb = plsc.unpack(s2, format=plsc.PackFormat.INTERLEAVED)
        da, db = plsc.unpack(d2, format=plsc.PackFormat.INTERLEAVED)
        t = sa * da + sb * db
        acc = t if acc is None else acc + t
    return acc


def _sc_body(nf_hbm, ei_hbm, out_hbm, mins_hbm, maxs_hbm,
             sidx_v, didx_v, srows_v, drows_v, labels_v, mm_v,
             sem_s, sem_d):
    sid = lax.axis_index("s")
    wid = sid * NC + lax.axis_index("c")
    base = wid * E_PER_W

    pltpu.sync_copy(ei_hbm.at[0, pl.ds(base, E_PER_W)], sidx_v)
    pltpu.sync_copy(ei_hbm.at[1, pl.ds(base, E_PER_W)], didx_v)

    def start_gather(c):
        b = c % NBUF
        pltpu.async_copy(
            nf_hbm.at[sidx_v.at[pl.ds(c * CHUNK, CHUNK)]],
            srows_v.at[b], sem_s)
        pltpu.async_copy(
            nf_hbm.at[didx_v.at[pl.ds(c * CHUNK, CHUNK)]],
            drows_v.at[b], sem_d)

    for cc in range(NBUF - 1):
        start_gather(cc)

    def chunk_body(c, carry):
        mn, mx = carry
        b = c % NBUF
        pltpu.make_async_copy(
            nf_hbm.at[sidx_v.at[pl.ds(c * CHUNK, CHUNK)]],
            srows_v.at[b], sem_s).wait()
        pltpu.make_async_copy(
            nf_hbm.at[didx_v.at[pl.ds(c * CHUNK, CHUNK)]],
            drows_v.at[b], sem_d).wait()

        @pl.when(c < NCHUNK - (NBUF - 1))
        def _():
            start_gather(c + NBUF - 1)

        lanes = lax.iota(jnp.int32, L)
        for g in range(GROUPS):
            lab = jnp.zeros((L,), jnp.float32)
            for e in range(L):
                acc = _edge_dot(srows_v.at[b], drows_v.at[b], g * L + e)
                for k in (8, 4, 2, 1):
                    acc = acc + _lane_shuffle(acc, lanes ^ k)
                lab = jnp.where(lanes == e, acc, lab)
            mn = jnp.minimum(mn, lab)
            mx = jnp.maximum(mx, lab)
            labels_v[pl.ds(c * CHUNK + g * L, L)] = lab
        return mn, mx

    init = (jnp.full((L,), jnp.inf, jnp.float32),
            jnp.full((L,), -jnp.inf, jnp.float32))
    mn, mx = lax.fori_loop(0, NCHUNK, chunk_body, init)

    mm_v[0, :] = mn
    mm_v[1, :] = mx
    pltpu.sync_copy(labels_v, out_hbm.at[pl.ds(base, E_PER_W)])
    pltpu.sync_copy(mm_v.at[0], mins_hbm.at[wid])
    pltpu.sync_copy(mm_v.at[1], maxs_hbm.at[wid])


def _pack_body(nf_ref, tbl_ref):
    u = lax.bitcast_convert_type(nf_ref[...], jnp.int32)
    r = u + 0x7FFF + (lax.shift_right_logical(u, 16) & 1)
    t = lax.shift_right_logical(r, 16)
    lo = t[:, : D // 2]
    hi = t[:, D // 2:]
    tbl_ref[...] = lax.shift_left(hi, 16) | lo


def _pack_table(nf):
    blk = N_NODES // 10
    return pl.pallas_call(
        _pack_body,
        grid=(10,),
        in_specs=[pl.BlockSpec((blk, D), lambda i: (i, 0))],
        out_specs=pl.BlockSpec((blk, D // 2), lambda i: (i, 0)),
        out_shape=jax.ShapeDtypeStruct((N_NODES, D // 2), jnp.int32),
    )(nf)


@jax.jit
def _sc_dot(nf_pack, edge_index):
    mesh = plsc.VectorSubcoreMesh(core_axis_name="c", subcore_axis_name="s")
    k = pl.kernel(
        _sc_body,
        out_type=(
            jax.ShapeDtypeStruct((N_EDGES,), jnp.float32),
            jax.ShapeDtypeStruct((NW, L), jnp.float32),
            jax.ShapeDtypeStruct((NW, L), jnp.float32),
        ),
        mesh=mesh,
        compiler_params=pltpu.CompilerParams(
            needs_layout_passes=False, use_tc_tiling_on_sc=False),
        scratch_types=[
            pltpu.VMEM((E_PER_W,), jnp.int32),
            pltpu.VMEM((E_PER_W,), jnp.int32),
            pltpu.VMEM((NBUF, CHUNK, D // 2), jnp.int32),
            pltpu.VMEM((NBUF, CHUNK, D // 2), jnp.int32),
            pltpu.VMEM((E_PER_W,), jnp.float32),
            pltpu.VMEM((2, L), jnp.float32),
            pltpu.SemaphoreType.DMA,
            pltpu.SemaphoreType.DMA,
        ],
    )
    return k(nf_pack, edge_index)


def _norm_body(lab_ref, mins_ref, maxs_ref, out_ref):
    mn = jnp.min(mins_ref[...])
    mx = jnp.max(maxs_ref[...])
    out_ref[...] = (lab_ref[...] - mn) / (mx - mn)


def _normalize(labels, mins, maxs):
    lab2 = labels.reshape(N_EDGES // D, D)
    return pl.pallas_call(
        _norm_body,
        out_shape=jax.ShapeDtypeStruct((N_EDGES // D, D), jnp.float32),
    )(lab2, mins.reshape(4, D), maxs.reshape(4, D))


def kernel(nf, edge_index):
    nf_pack = _pack_table(nf)
    labels, mins, maxs = _sc_dot(nf_pack, edge_index)
    out = _normalize(labels, mins, maxs)
    return out.reshape(N_EDGES, 1)


# speedup vs baseline: 1.0735x; 1.0186x over previous
import functools

import jax
import jax.numpy as jnp
from jax import lax
from jax.experimental import pallas as pl
from jax.experimental.pallas import tpu as pltpu
from jax.experimental.pallas import tpu_sc as plsc

N_NODES = 10000
N_EDGES = 320000
D = 128
L = 16
NC = 2
NS = 16
NW = NC * NS
E_PER_W = N_EDGES // NW
CHUNK = 80
NCHUNK = E_PER_W // CHUNK
GROUPS = CHUNK // L
NBUF = 8


def _lane_shuffle(v, idx):
    dnums = lax.GatherDimensionNumbers(
        offset_dims=(), collapsed_slice_dims=(0,), start_index_map=(0,))
    return lax.gather(v, idx[:, None], dnums, slice_sizes=(1,),
                      mode=lax.GatherScatterMode.PROMISE_IN_BOUNDS)


def _edge_dot(srows_v, drows_v, row):
    acc = None
    for j in range(D // 32):
        s2 = plsc.bitcast(srows_v[row, pl.ds(j * L, L)], jnp.bfloat16)
        d2 = plsc.bitcast(drows_v[row, pl.ds(j * L, L)], jnp.bfloat16)
        sa, sb = plsc.unpack(s2, format=plsc.PackFormat.INTERLEAVED)
        da, db = plsc.unpack(d2, format=plsc.PackFormat.INTERLEAVED)
        t = sa * da + sb * db
        acc = t if acc is None else acc + t
    return acc


def _sc_body(nf_hbm, ei_hbm, out_hbm, mins_hbm, maxs_hbm,
             sidx_v, didx_v, srows_v, drows_v, labels_v, mm_v,
             sem_s, sem_d):
    sid = lax.axis_index("s")
    wid = sid * NC + lax.axis_index("c")
    base = wid * E_PER_W

    pltpu.sync_copy(ei_hbm.at[0, pl.ds(base, E_PER_W)], sidx_v)
    pltpu.sync_copy(ei_hbm.at[1, pl.ds(base, E_PER_W)], didx_v)

    def start_gather(c):
        b = c % NBUF
        pltpu.async_copy(
            nf_hbm.at[sidx_v.at[pl.ds(c * CHUNK, CHUNK)]],
            srows_v.at[b], sem_s)
        pltpu.async_copy(
            nf_hbm.at[didx_v.at[pl.ds(c * CHUNK, CHUNK)]],
            drows_v.at[b], sem_d)

    for cc in range(NBUF - 1):
        start_gather(cc)

    def chunk_body(c, carry):
        mn, mx = carry
        b = c % NBUF
        pltpu.make_async_copy(
            nf_hbm.at[sidx_v.at[pl.ds(c * CHUNK, CHUNK)]],
            srows_v.at[b], sem_s).wait()
        pltpu.make_async_copy(
            nf_hbm.at[didx_v.at[pl.ds(c * CHUNK, CHUNK)]],
            drows_v.at[b], sem_d).wait()

        @pl.when(c < NCHUNK - (NBUF - 1))
        def _():
            start_gather(c + NBUF - 1)

        lanes = lax.iota(jnp.int32, L)
        for g in range(GROUPS):
            lab = jnp.zeros((L,), jnp.float32)
            for e in range(L):
                acc = _edge_dot(srows_v.at[b], drows_v.at[b], g * L + e)
                for k in (8, 4, 2, 1):
                    acc = acc + _lane_shuffle(acc, lanes ^ k)
                lab = jnp.where(lanes == e, acc, lab)
            mn = jnp.minimum(mn, lab)
            mx = jnp.maximum(mx, lab)
            labels_v[pl.ds(c * CHUNK + g * L, L)] = lab
        return mn, mx

    init = (jnp.full((L,), jnp.inf, jnp.float32),
            jnp.full((L,), -jnp.inf, jnp.float32))
    mn, mx = lax.fori_loop(0, NCHUNK, chunk_body, init)

    mm_v[0, :] = mn
    mm_v[1, :] = mx
    pltpu.sync_copy(labels_v, out_hbm.at[pl.ds(base, E_PER_W)])
    pltpu.sync_copy(mm_v.at[0], mins_hbm.at[wid])
    pltpu.sync_copy(mm_v.at[1], maxs_hbm.at[wid])


def _pack_body(nf_ref, tbl_ref):
    u = lax.bitcast_convert_type(nf_ref[...], jnp.int32)
    r = u + 0x7FFF + (lax.shift_right_logical(u, 16) & 1)
    t = lax.shift_right_logical(r, 16)
    lo = t[:, : D // 2]
    hi = t[:, D // 2:]
    tbl_ref[...] = lax.shift_left(hi, 16) | lo


def _pack_table(nf):
    return pl.pallas_call(
        _pack_body,
        out_shape=jax.ShapeDtypeStruct((N_NODES, D // 2), jnp.int32),
    )(nf)


@jax.jit
def _sc_dot(nf_pack, edge_index):
    mesh = plsc.VectorSubcoreMesh(core_axis_name="c", subcore_axis_name="s")
    k = pl.kernel(
        _sc_body,
        out_type=(
            jax.ShapeDtypeStruct((N_EDGES,), jnp.float32),
            jax.ShapeDtypeStruct((NW, L), jnp.float32),
            jax.ShapeDtypeStruct((NW, L), jnp.float32),
        ),
        mesh=mesh,
        compiler_params=pltpu.CompilerParams(
            needs_layout_passes=False, use_tc_tiling_on_sc=False),
        scratch_types=[
            pltpu.VMEM((E_PER_W,), jnp.int32),
            pltpu.VMEM((E_PER_W,), jnp.int32),
            pltpu.VMEM((NBUF, CHUNK, D // 2), jnp.int32),
            pltpu.VMEM((NBUF, CHUNK, D // 2), jnp.int32),
            pltpu.VMEM((E_PER_W,), jnp.float32),
            pltpu.VMEM((2, L), jnp.float32),
            pltpu.SemaphoreType.DMA,
            pltpu.SemaphoreType.DMA,
        ],
    )
    return k(nf_pack, edge_index)


def _norm_body(lab_ref, mins_ref, maxs_ref, out_ref):
    mn = jnp.min(mins_ref[...])
    mx = jnp.max(maxs_ref[...])
    out_ref[...] = (lab_ref[...] - mn) / (mx - mn)


def _normalize(labels, mins, maxs):
    lab2 = labels.reshape(N_EDGES // D, D)
    return pl.pallas_call(
        _norm_body,
        out_shape=jax.ShapeDtypeStruct((N_EDGES // D, D), jnp.float32),
    )(lab2, mins.reshape(4, D), maxs.reshape(4, D))


def kernel(nf, edge_index):
    nf_pack = _pack_table(nf)
    labels, mins, maxs = _sc_dot(nf_pack, edge_index)
    out = _normalize(labels, mins, maxs)
    return out.reshape(N_EDGES, 1)


# speedup vs baseline: 1.0739x; 1.0003x over previous
import jax
import jax.numpy as jnp
from jax import lax
from jax.experimental import pallas as pl
from jax.experimental.pallas import tpu as pltpu
from jax.experimental.pallas import tpu_sc as plsc

N_NODES = 10000
N_EDGES = 320000
D = 128
L = 16
NC = 2
NS = 16
NW = NC * NS
E_PER_W = N_EDGES // NW
CHUNK = 80
NCHUNK = E_PER_W // CHUNK
GROUPS = CHUNK // L
NBUF = 8


def _lane_shuffle(v, idx):
    dnums = lax.GatherDimensionNumbers(
        offset_dims=(), collapsed_slice_dims=(0,), start_index_map=(0,))
    return lax.gather(v, idx[:, None], dnums, slice_sizes=(1,),
                      mode=lax.GatherScatterMode.PROMISE_IN_BOUNDS)


def _edge_dot(srows_v, drows_v, row):
    acc = None
    for j in range(D // 32):
        s2 = plsc.bitcast(srows_v[row, pl.ds(j * L, L)], jnp.bfloat16)
        d2 = plsc.bitcast(drows_v[row, pl.ds(j * L, L)], jnp.bfloat16)
        sa, sb = plsc.unpack(s2, format=plsc.PackFormat.INTERLEAVED)
        da, db = plsc.unpack(d2, format=plsc.PackFormat.INTERLEAVED)
        t = sa * da + sb * db
        acc = t if acc is None else acc + t
    return acc


def _sc_body(nf_hbm, ei_hbm, out_hbm, mins_hbm, maxs_hbm,
             sidx_v, didx_v, srows_v, drows_v, labels_v, mm_v,
             sem_s, sem_d):
    sid = lax.axis_index("s")
    wid = sid * NC + lax.axis_index("c")
    base = wid * E_PER_W

    pltpu.sync_copy(ei_hbm.at[0, pl.ds(base, E_PER_W)], sidx_v)
    pltpu.sync_copy(ei_hbm.at[1, pl.ds(base, E_PER_W)], didx_v)

    def start_gather(c):
        b = c % NBUF
        pltpu.async_copy(
            nf_hbm.at[sidx_v.at[pl.ds(c * CHUNK, CHUNK)]],
            srows_v.at[b], sem_s)
        pltpu.async_copy(
            nf_hbm.at[didx_v.at[pl.ds(c * CHUNK, CHUNK)]],
            drows_v.at[b], sem_d)

    for cc in range(NBUF - 1):
        start_gather(cc)

    def chunk_body(c, carry):
        mn, mx = carry
        b = c % NBUF
        pltpu.make_async_copy(
            nf_hbm.at[sidx_v.at[pl.ds(c * CHUNK, CHUNK)]],
            srows_v.at[b], sem_s).wait()
        pltpu.make_async_copy(
            nf_hbm.at[didx_v.at[pl.ds(c * CHUNK, CHUNK)]],
            drows_v.at[b], sem_d).wait()

        @pl.when(c < NCHUNK - (NBUF - 1))
        def _():
            start_gather(c + NBUF - 1)

        lanes = lax.iota(jnp.int32, L)
        for g in range(GROUPS):
            lab = jnp.zeros((L,), jnp.float32)
            for e in range(L):
                acc = _edge_dot(srows_v.at[b], drows_v.at[b], g * L + e)
                for k in (8, 4, 2, 1):
                    acc = acc + _lane_shuffle(acc, lanes ^ k)
                lab = jnp.where(lanes == e, acc, lab)
            mn = jnp.minimum(mn, lab)
            mx = jnp.maximum(mx, lab)
            labels_v[pl.ds(c * CHUNK + g * L, L)] = lab
        return mn, mx

    init = (jnp.full((L,), jnp.inf, jnp.float32),
            jnp.full((L,), -jnp.inf, jnp.float32))
    mn, mx = lax.fori_loop(0, NCHUNK, chunk_body, init)

    mm_v[0, :] = mn
    mm_v[1, :] = mx
    pltpu.sync_copy(labels_v, out_hbm.at[pl.ds(base, E_PER_W)])
    pltpu.sync_copy(mm_v.at[0], mins_hbm.at[wid])
    pltpu.sync_copy(mm_v.at[1], maxs_hbm.at[wid])


def _pack_body(nf_ref, tbl_ref):
    u = lax.bitcast_convert_type(nf_ref[...], jnp.int32)
    r = u + 0x7FFF + (lax.shift_right_logical(u, 16) & 1)
    t = lax.shift_right_logical(r, 16)
    lo = t[:, : D // 2]
    hi = t[:, D // 2:]
    tbl_ref[...] = lax.shift_left(hi, 16) | lo


def _pack_table(nf):
    return pl.pallas_call(
        _pack_body,
        out_shape=jax.ShapeDtypeStruct((N_NODES, D // 2), jnp.int32),
    )(nf)


@jax.jit
def _sc_dot(nf_pack, edge_index):
    mesh = plsc.VectorSubcoreMesh(core_axis_name="c", subcore_axis_name="s")
    k = pl.kernel(
        _sc_body,
        out_type=(
            jax.ShapeDtypeStruct((N_EDGES,), jnp.float32),
            jax.ShapeDtypeStruct((NW, L), jnp.float32),
            jax.ShapeDtypeStruct((NW, L), jnp.float32),
        ),
        mesh=mesh,
        compiler_params=pltpu.CompilerParams(
            needs_layout_passes=False, use_tc_tiling_on_sc=False),
        scratch_types=[
            pltpu.VMEM((E_PER_W,), jnp.int32),
            pltpu.VMEM((E_PER_W,), jnp.int32),
            pltpu.VMEM((NBUF, CHUNK, D // 2), jnp.int32),
            pltpu.VMEM((NBUF, CHUNK, D // 2), jnp.int32),
            pltpu.VMEM((E_PER_W,), jnp.float32),
            pltpu.VMEM((2, L), jnp.float32),
            pltpu.SemaphoreType.DMA,
            pltpu.SemaphoreType.DMA,
        ],
    )
    return k(nf_pack, edge_index)


def _norm_body(lab_ref, mins_ref, maxs_ref, out_ref):
    mn = jnp.min(mins_ref[...])
    mx = jnp.max(maxs_ref[...])
    out_ref[...] = (lab_ref[...] - mn) / (mx - mn)


def _normalize(labels, mins, maxs):
    lab2 = labels.reshape(N_EDGES // D, D)
    return pl.pallas_call(
        _norm_body,
        out_shape=jax.ShapeDtypeStruct((N_EDGES // D, D), jnp.float32),
    )(lab2, mins.reshape(4, D), maxs.reshape(4, D))


def kernel(nf, edge_index):
    nf_pack = _pack_table(nf)
    labels, mins, maxs = _sc_dot(nf_pack, edge_index)
    out = _normalize(labels, mins, maxs)
    return out.reshape(N_EDGES, 1)
